# baseline (speedup 1.0000x reference)
import jax
import jax.numpy as jnp
from jax import lax
from jax.experimental import pallas as pl
from jax.experimental.pallas import tpu as pltpu
from jax.experimental.pallas import tpu_sc as plsc

_N = 1024
_D = 128
_NPAIR = 128
_L = 16
_NJB = _N // _L
_BUF = 1280
_NW = 16
_PW = _NPAIR // _NW
_MAGIC = 1597463007


def _rsqrt(x):
    y = plsc.bitcast(_MAGIC - (plsc.bitcast(x, jnp.int32) >> 1), jnp.float32)
    for _ in range(4):
        y = y * (1.5 - 0.5 * x * y * y)
    return y


def _sc_loss(emb_hbm, label_hbm, out_hbm,
             label_v, pi_v, ni_v,
             idx_sh, part_sh, idx_pk_v, idx_v,
             pa_v, pb_v, na_v, nb_v,
             part_v, parts_v, outv_v, cnt_s, sem):
    cid = lax.axis_index("c")
    sid = lax.axis_index("s")
    w0 = (cid == 0) & (sid == 0)
    iota = lax.broadcasted_iota(jnp.int32, (_L,), 0)

    @pl.when(w0)
    def _select():
        pltpu.sync_copy(label_hbm, label_v)

        for b in range(_NPAIR // _L + 1):
            sl = pl.ds(b * _L, _L)
            pi_v[sl] = jnp.full((_L,), (_N - 1) * _N, jnp.int32)
            ni_v[sl] = jnp.full((_L,), (_N - 1) * _N, jnp.int32)

        cnt_s[0] = jnp.int32(0)
        cnt_s[1] = jnp.int32(0)

        def row_work(i):
            pp = cnt_s[0]
            pn = cnt_s[1]

            @pl.when((pp < _NPAIR) | (pn < _NPAIR))
            def _row():
                ivec = jnp.full((_L,), i, jnp.int32)
                lab_i = plsc.load_gather(label_v, [ivec])
                do_p = pp < _NPAIR
                do_n = pn < _NPAIR

                ipacked = jnp.full((_L,), i * _N, jnp.int32)

                def jb_body(b, c2):
                    pp2, pn2 = c2
                    base = b * _L
                    labs = label_v[pl.ds(base, _L)]
                    jvec = base + iota
                    valid = jvec > i
                    eq = labs == lab_i
                    pmr = valid & eq
                    nm = valid & jnp.logical_not(eq) & do_n
                    packed = ipacked + jvec
                    cntp = plsc.all_reduce_population_count(pmr)[0]
                    cntv = jnp.minimum(_L, base + (_L - 1) - i)
                    vrank = jnp.minimum(jvec - ivec, iota + 1)

                    @pl.when(cntp > 0)
                    def _with_pos():
                        rp = plsc.cumsum(pmr.astype(jnp.int32))
                        plsc.store_scatter(pi_v, [pp2 + rp - 1], packed,
                                           mask=pmr & do_p)
                        plsc.store_scatter(ni_v, [pn2 + (vrank - rp) - 1],
                                           packed, mask=nm)

                    @pl.when((cntp == 0) & do_n)
                    def _neg_only():
                        plsc.store_scatter(ni_v, [pn2 + vrank - 1], packed,
                                           mask=nm)

                    pp2 = pp2 + jnp.where(do_p, cntp, 0)
                    pn2 = pn2 + jnp.where(do_n, cntv - cntp, 0)
                    return (pp2, pn2)

                pp2, pn2 = lax.fori_loop((i + 1) // _L, _NJB, jb_body, (pp, pn))
                cnt_s[0] = pp2
                cnt_s[1] = pn2

        def grp_body(g, carry):
            pp = cnt_s[0]
            pn = cnt_s[1]

            @pl.when((pp < _NPAIR) | (pn < _NPAIR))
            def _grp():
                def row_body(t, c):
                    i = g * _L + t

                    @pl.when(i < _N - 1)
                    def _():
                        row_work(i)

                    return c

                lax.fori_loop(0, _L, row_body, jnp.int32(0))

            return carry

        lax.fori_loop(0, _NJB, grp_body, jnp.int32(0))

        pltpu.sync_copy(pi_v.at[pl.ds(0, _NPAIR)], idx_sh.at[pl.ds(0, _NPAIR)])
        pltpu.sync_copy(ni_v.at[pl.ds(0, _NPAIR)],
                        idx_sh.at[pl.ds(_NPAIR, _NPAIR)])

    plsc.subcore_barrier()

    @pl.when(cid == 0)
    def _compute():
        pltpu.sync_copy(idx_sh, idx_pk_v)
        for b in range(_NPAIR // _L):
            pk = idx_pk_v[pl.ds(b * _L, _L)]
            idx_v[pl.ds(b * _L, _L)] = pk >> 10
            idx_v[pl.ds(_NPAIR + b * _L, _L)] = pk & (_N - 1)
            nk = idx_pk_v[pl.ds(_NPAIR + b * _L, _L)]
            idx_v[pl.ds(2 * _NPAIR + b * _L, _L)] = nk >> 10
            idx_v[pl.ds(3 * _NPAIR + b * _L, _L)] = nk & (_N - 1)
        base = sid * _PW
        cpa = pltpu.async_copy(emb_hbm.at[idx_v.at[pl.ds(base, _PW)]],
                               pa_v, sem)
        cpb = pltpu.async_copy(emb_hbm.at[idx_v.at[pl.ds(_NPAIR + base, _PW)]],
                               pb_v, sem)
        cna = pltpu.async_copy(
            emb_hbm.at[idx_v.at[pl.ds(2 * _NPAIR + base, _PW)]], na_v, sem)
        cnb = pltpu.async_copy(
            emb_hbm.at[idx_v.at[pl.ds(3 * _NPAIR + base, _PW)]], nb_v, sem)
        cpa.wait()
        cpb.wait()
        cna.wait()
        cnb.wait()

        def pos_body(p, acc):
            for v in range(_D // _L):
                t = pa_v[p, pl.ds(v * _L, _L)] - pb_v[p, pl.ds(v * _L, _L)] + 1e-6
                acc = acc + t * t
            return acc

        acc = lax.fori_loop(0, _PW, pos_body, jnp.zeros((_L,), jnp.float32))

        def neg_body(p, d2g):
            def kv(v, a):
                t = na_v[p, pl.ds(v * _L, _L)] - nb_v[p, pl.ds(v * _L, _L)] + 1e-6
                return a + t * t

            a = lax.fori_loop(0, _D // _L, kv, jnp.zeros((_L,), jnp.float32))
            return jnp.where(iota == p, jnp.sum(a), d2g)

        d2g = lax.fori_loop(0, _PW, neg_body, jnp.zeros((_L,), jnp.float32))
        d = d2g * _rsqrt(jnp.maximum(d2g, 1e-30))
        t = jnp.maximum(2.0 - d, 0.0)
        acc = acc + jnp.where(iota < _PW, t * t, 0.0)

        part_v[...] = acc
        pltpu.sync_copy(part_v, part_sh.at[sid])

    plsc.subcore_barrier()

    @pl.when(w0)
    def _reduce():
        pltpu.sync_copy(part_sh, parts_v)
        total_v = jnp.zeros((_L,), jnp.float32)
        for w in range(_NW):
            total_v = total_v + parts_v[w, pl.ds(0, _L)]
        total = jnp.sum(total_v)
        outv_v[...] = jnp.where(iota == 0, total, total * (1.0 / (2 * _NPAIR)))
        pltpu.sync_copy(outv_v, out_hbm)


@jax.jit
def kernel(embedding, label):
    out = pl.kernel(
        _sc_loss,
        out_type=jax.ShapeDtypeStruct((_L,), jnp.float32),
        mesh=plsc.VectorSubcoreMesh(core_axis_name="c", subcore_axis_name="s"),
        compiler_params=pltpu.CompilerParams(needs_layout_passes=False),
        scratch_types=[
            pltpu.VMEM((_N,), jnp.int32),
            pltpu.VMEM((_BUF,), jnp.int32),
            pltpu.VMEM((_BUF,), jnp.int32),
            pltpu.VMEM_SHARED((2 * _NPAIR,), jnp.int32),
            pltpu.VMEM_SHARED((_NW, _L), jnp.float32),
            pltpu.VMEM((2 * _NPAIR,), jnp.int32),
            pltpu.VMEM((4 * _NPAIR,), jnp.int32),
            pltpu.VMEM((_PW, _D), jnp.float32),
            pltpu.VMEM((_PW, _D), jnp.float32),
            pltpu.VMEM((_PW, _D), jnp.float32),
            pltpu.VMEM((_PW, _D), jnp.float32),
            pltpu.VMEM((_L,), jnp.float32),
            pltpu.VMEM((_NW, _L), jnp.float32),
            pltpu.VMEM((_L,), jnp.float32),
            pltpu.SMEM((2,), jnp.int32),
            pltpu.SemaphoreType.DMA,
        ],
    )(embedding, label)
    return (out[1], out[0], (2 * _NPAIR,))


# speedup vs baseline: 1.2635x; 1.2635x over previous
import jax
import jax.numpy as jnp
from jax import lax
from jax.experimental import pallas as pl
from jax.experimental.pallas import tpu as pltpu
from jax.experimental.pallas import tpu_sc as plsc

_N = 1024
_D = 128
_NPAIR = 128
_L = 16
_NJB = _N // _L
_BUF = 1280
_NW = 16
_PW = _NPAIR // _NW
_MAGIC = 1597463007


def _rsqrt(x):
    y = plsc.bitcast(_MAGIC - (plsc.bitcast(x, jnp.int32) >> 1), jnp.float32)
    for _ in range(4):
        y = y * (1.5 - 0.5 * x * y * y)
    return y


def _sc_loss(emb_hbm, label_hbm, out_hbm,
             label_v, pi_v, ni_v,
             idx_sh, part_sh, idx_pk_v, idx_v,
             pa_v, pb_v, na_v, nb_v,
             part_v, parts_v, outv_v, cnt_s, sem):
    cid = lax.axis_index("c")
    sid = lax.axis_index("s")
    w0 = (cid == 0) & (sid == 0)
    iota = lax.broadcasted_iota(jnp.int32, (_L,), 0)

    @pl.when(w0)
    def _select():
        pltpu.sync_copy(label_hbm, label_v)

        for b in range(_NPAIR // _L + 1):
            sl = pl.ds(b * _L, _L)
            pi_v[sl] = jnp.full((_L,), (_N - 1) * _N, jnp.int32)
            ni_v[sl] = jnp.full((_L,), (_N - 1) * _N, jnp.int32)

        cnt_s[0] = jnp.int32(0)
        cnt_s[1] = jnp.int32(0)

        def row_work(i):
            pp = cnt_s[0]
            pn = cnt_s[1]

            @pl.when((pp < _NPAIR) | (pn < _NPAIR))
            def _row():
                ivec = jnp.full((_L,), i, jnp.int32)
                lab_i = plsc.load_gather(label_v, [ivec])
                do_p = pp < _NPAIR
                do_n = pn < _NPAIR

                ipacked = jnp.full((_L,), i * _N, jnp.int32)

                def jb_body(b, c2):
                    pp2, pn2 = c2
                    base = b * _L
                    labs = label_v[pl.ds(base, _L)]
                    jvec = base + iota
                    valid = jvec > i
                    eq = labs == lab_i
                    pmr = valid & eq
                    nm = valid & jnp.logical_not(eq) & do_n
                    packed = ipacked + jvec
                    cntv = jnp.minimum(_L, base + (_L - 1) - i)
                    rp = plsc.cumsum(pmr.astype(jnp.int32))
                    plsc.store_scatter(pi_v, [pp2 + rp - 1], packed,
                                       mask=pmr & do_p)
                    rn = jnp.minimum(jvec - ivec, iota + 1) - rp
                    plsc.store_scatter(ni_v, [pn2 + rn - 1], packed, mask=nm)
                    cntp = rp[_L - 1]
                    pp2 = pp2 + jnp.where(do_p, cntp, 0)
                    pn2 = pn2 + jnp.where(do_n, cntv - cntp, 0)
                    return (pp2, pn2)

                pp2, pn2 = lax.fori_loop((i + 1) // _L, _NJB, jb_body, (pp, pn))
                cnt_s[0] = pp2
                cnt_s[1] = pn2

        def grp_body(g, carry):
            pp = cnt_s[0]
            pn = cnt_s[1]

            @pl.when((pp < _NPAIR) | (pn < _NPAIR))
            def _grp():
                def row_body(t, c):
                    i = g * _L + t

                    @pl.when(i < _N - 1)
                    def _():
                        row_work(i)

                    return c

                lax.fori_loop(0, _L, row_body, jnp.int32(0))

            return carry

        lax.fori_loop(0, _NJB, grp_body, jnp.int32(0))

        pltpu.sync_copy(pi_v.at[pl.ds(0, _NPAIR)], idx_sh.at[pl.ds(0, _NPAIR)])
        pltpu.sync_copy(ni_v.at[pl.ds(0, _NPAIR)],
                        idx_sh.at[pl.ds(_NPAIR, _NPAIR)])

    plsc.subcore_barrier()

    @pl.when(cid == 0)
    def _compute():
        pltpu.sync_copy(idx_sh, idx_pk_v)
        for b in range(_NPAIR // _L):
            pk = idx_pk_v[pl.ds(b * _L, _L)]
            idx_v[pl.ds(b * _L, _L)] = pk >> 10
            idx_v[pl.ds(_NPAIR + b * _L, _L)] = pk & (_N - 1)
            nk = idx_pk_v[pl.ds(_NPAIR + b * _L, _L)]
            idx_v[pl.ds(2 * _NPAIR + b * _L, _L)] = nk >> 10
            idx_v[pl.ds(3 * _NPAIR + b * _L, _L)] = nk & (_N - 1)
        base = sid * _PW
        cpa = pltpu.async_copy(emb_hbm.at[idx_v.at[pl.ds(base, _PW)]],
                               pa_v, sem)
        cpb = pltpu.async_copy(emb_hbm.at[idx_v.at[pl.ds(_NPAIR + base, _PW)]],
                               pb_v, sem)
        cna = pltpu.async_copy(
            emb_hbm.at[idx_v.at[pl.ds(2 * _NPAIR + base, _PW)]], na_v, sem)
        cnb = pltpu.async_copy(
            emb_hbm.at[idx_v.at[pl.ds(3 * _NPAIR + base, _PW)]], nb_v, sem)
        cpa.wait()
        cpb.wait()
        cna.wait()
        cnb.wait()

        def pos_body(p, acc):
            for v in range(_D // _L):
                t = pa_v[p, pl.ds(v * _L, _L)] - pb_v[p, pl.ds(v * _L, _L)] + 1e-6
                acc = acc + t * t
            return acc

        acc = lax.fori_loop(0, _PW, pos_body, jnp.zeros((_L,), jnp.float32))

        def neg_body(p, d2g):
            def kv(v, a):
                t = na_v[p, pl.ds(v * _L, _L)] - nb_v[p, pl.ds(v * _L, _L)] + 1e-6
                return a + t * t

            a = lax.fori_loop(0, _D // _L, kv, jnp.zeros((_L,), jnp.float32))
            return jnp.where(iota == p, jnp.sum(a), d2g)

        d2g = lax.fori_loop(0, _PW, neg_body, jnp.zeros((_L,), jnp.float32))
        d = d2g * _rsqrt(jnp.maximum(d2g, 1e-30))
        t = jnp.maximum(2.0 - d, 0.0)
        acc = acc + jnp.where(iota < _PW, t * t, 0.0)

        part_v[...] = acc
        pltpu.sync_copy(part_v, part_sh.at[sid])

    plsc.subcore_barrier()

    @pl.when(w0)
    def _reduce():
        pltpu.sync_copy(part_sh, parts_v)
        total_v = jnp.zeros((_L,), jnp.float32)
        for w in range(_NW):
            total_v = total_v + parts_v[w, pl.ds(0, _L)]
        total = jnp.sum(total_v)
        outv_v[...] = jnp.where(iota == 0, total, total * (1.0 / (2 * _NPAIR)))
        pltpu.sync_copy(outv_v, out_hbm)


@jax.jit
def kernel(embedding, label):
    out = pl.kernel(
        _sc_loss,
        out_type=jax.ShapeDtypeStruct((_L,), jnp.float32),
        mesh=plsc.VectorSubcoreMesh(core_axis_name="c", subcore_axis_name="s"),
        compiler_params=pltpu.CompilerParams(needs_layout_passes=False),
        scratch_types=[
            pltpu.VMEM((_N,), jnp.int32),
            pltpu.VMEM((_BUF,), jnp.int32),
            pltpu.VMEM((_BUF,), jnp.int32),
            pltpu.VMEM_SHARED((2 * _NPAIR,), jnp.int32),
            pltpu.VMEM_SHARED((_NW, _L), jnp.float32),
            pltpu.VMEM((2 * _NPAIR,), jnp.int32),
            pltpu.VMEM((4 * _NPAIR,), jnp.int32),
            pltpu.VMEM((_PW, _D), jnp.float32),
            pltpu.VMEM((_PW, _D), jnp.float32),
            pltpu.VMEM((_PW, _D), jnp.float32),
            pltpu.VMEM((_PW, _D), jnp.float32),
            pltpu.VMEM((_L,), jnp.float32),
            pltpu.VMEM((_NW, _L), jnp.float32),
            pltpu.VMEM((_L,), jnp.float32),
            pltpu.SMEM((2,), jnp.int32),
            pltpu.SemaphoreType.DMA,
        ],
    )(embedding, label)
    return (out[1], out[0], (2 * _NPAIR,))


# speedup vs baseline: 1.4454x; 1.1440x over previous
import jax
import jax.numpy as jnp
from jax import lax
from jax.experimental import pallas as pl
from jax.experimental.pallas import tpu as pltpu
from jax.experimental.pallas import tpu_sc as plsc

_N = 1024
_D = 128
_NPAIR = 128
_L = 16
_NJB = _N // _L
_BUF = 1280
_NW = 16
_PW = _NPAIR // _NW
_MAGIC = 1597463007


def _rsqrt(x):
    y = plsc.bitcast(_MAGIC - (plsc.bitcast(x, jnp.int32) >> 1), jnp.float32)
    for _ in range(4):
        y = y * (1.5 - 0.5 * x * y * y)
    return y


def _sc_loss(emb_hbm, label_hbm, out_hbm,
             label_v, pi_v, ni_v,
             idx_sh, part_sh, idx_pk_v, idx_v,
             pa_v, pb_v, na_v, nb_v,
             part_v, parts_v, outv_v, cnt_s, sem):
    cid = lax.axis_index("c")
    sid = lax.axis_index("s")
    w0 = (cid == 0) & (sid == 0)
    iota = lax.broadcasted_iota(jnp.int32, (_L,), 0)

    @pl.when(w0)
    def _select():
        pltpu.sync_copy(label_hbm, label_v)

        for b in range(_NPAIR // _L + 1):
            sl = pl.ds(b * _L, _L)
            pi_v[sl] = jnp.full((_L,), (_N - 1) * _N, jnp.int32)
            ni_v[sl] = jnp.full((_L,), (_N - 1) * _N, jnp.int32)

        cnt_s[0] = jnp.int32(0)
        cnt_s[1] = jnp.int32(0)

        def row_work(i):
            pp = cnt_s[0]
            pn = cnt_s[1]

            @pl.when((pp < _NPAIR) | (pn < _NPAIR))
            def _row():
                ivec = jnp.full((_L,), i, jnp.int32)
                lab_i = plsc.load_gather(label_v, [ivec])
                do_p = pp < _NPAIR
                do_n = pn < _NPAIR

                ipacked = jnp.full((_L,), i * _N, jnp.int32)

                @plsc.parallel_loop((i + 1) // _L, _NJB, unroll=4,
                                    carry=(pp, pn))
                def jb_body(b, c2):
                    pp2, pn2 = c2
                    base = b * _L
                    labs = label_v[pl.ds(base, _L)]
                    jvec = base + iota
                    valid = jvec > i
                    eq = labs == lab_i
                    pmr = valid & eq
                    nm = valid & jnp.logical_not(eq) & do_n
                    packed = ipacked + jvec
                    cntv = jnp.minimum(_L, base + (_L - 1) - i)
                    cntp = plsc.all_reduce_population_count(pmr)[0]
                    rp = plsc.cumsum(pmr.astype(jnp.int32))
                    plsc.store_scatter(pi_v, [pp2 + rp - 1], packed,
                                       mask=pmr & do_p)
                    rn = jnp.minimum(jvec - ivec, iota + 1) - rp
                    plsc.store_scatter(ni_v, [pn2 + rn - 1], packed, mask=nm)
                    pp2 = pp2 + jnp.where(do_p, cntp, 0)
                    pn2 = pn2 + jnp.where(do_n, cntv - cntp, 0)
                    return (pp2, pn2)

                pp2, pn2 = jb_body
                cnt_s[0] = pp2
                cnt_s[1] = pn2

        def grp_body(g, carry):
            pp = cnt_s[0]
            pn = cnt_s[1]

            @pl.when((pp < _NPAIR) | (pn < _NPAIR))
            def _grp():
                def row_body(t, c):
                    i = g * _L + t

                    @pl.when(i < _N - 1)
                    def _():
                        row_work(i)

                    return c

                lax.fori_loop(0, _L, row_body, jnp.int32(0))

            return carry

        lax.fori_loop(0, _NJB, grp_body, jnp.int32(0))

        pltpu.sync_copy(pi_v.at[pl.ds(0, _NPAIR)], idx_sh.at[pl.ds(0, _NPAIR)])
        pltpu.sync_copy(ni_v.at[pl.ds(0, _NPAIR)],
                        idx_sh.at[pl.ds(_NPAIR, _NPAIR)])

    plsc.subcore_barrier()

    @pl.when(cid == 0)
    def _compute():
        pltpu.sync_copy(idx_sh, idx_pk_v)
        for b in range(_NPAIR // _L):
            pk = idx_pk_v[pl.ds(b * _L, _L)]
            idx_v[pl.ds(b * _L, _L)] = pk >> 10
            idx_v[pl.ds(_NPAIR + b * _L, _L)] = pk & (_N - 1)
            nk = idx_pk_v[pl.ds(_NPAIR + b * _L, _L)]
            idx_v[pl.ds(2 * _NPAIR + b * _L, _L)] = nk >> 10
            idx_v[pl.ds(3 * _NPAIR + b * _L, _L)] = nk & (_N - 1)
        base = sid * _PW
        cpa = pltpu.async_copy(emb_hbm.at[idx_v.at[pl.ds(base, _PW)]],
                               pa_v, sem)
        cpb = pltpu.async_copy(emb_hbm.at[idx_v.at[pl.ds(_NPAIR + base, _PW)]],
                               pb_v, sem)
        cna = pltpu.async_copy(
            emb_hbm.at[idx_v.at[pl.ds(2 * _NPAIR + base, _PW)]], na_v, sem)
        cnb = pltpu.async_copy(
            emb_hbm.at[idx_v.at[pl.ds(3 * _NPAIR + base, _PW)]], nb_v, sem)
        cpa.wait()
        cpb.wait()
        cna.wait()
        cnb.wait()

        def pos_body(p, acc):
            for v in range(_D // _L):
                t = pa_v[p, pl.ds(v * _L, _L)] - pb_v[p, pl.ds(v * _L, _L)] + 1e-6
                acc = acc + t * t
            return acc

        acc = lax.fori_loop(0, _PW, pos_body, jnp.zeros((_L,), jnp.float32))

        def neg_body(p, d2g):
            def kv(v, a):
                t = na_v[p, pl.ds(v * _L, _L)] - nb_v[p, pl.ds(v * _L, _L)] + 1e-6
                return a + t * t

            a = lax.fori_loop(0, _D // _L, kv, jnp.zeros((_L,), jnp.float32))
            return jnp.where(iota == p, jnp.sum(a), d2g)

        d2g = lax.fori_loop(0, _PW, neg_body, jnp.zeros((_L,), jnp.float32))
        d = d2g * _rsqrt(jnp.maximum(d2g, 1e-30))
        t = jnp.maximum(2.0 - d, 0.0)
        acc = acc + jnp.where(iota < _PW, t * t, 0.0)

        part_v[...] = acc
        pltpu.sync_copy(part_v, part_sh.at[sid])

    plsc.subcore_barrier()

    @pl.when(w0)
    def _reduce():
        pltpu.sync_copy(part_sh, parts_v)
        total_v = jnp.zeros((_L,), jnp.float32)
        for w in range(_NW):
            total_v = total_v + parts_v[w, pl.ds(0, _L)]
        total = jnp.sum(total_v)
        outv_v[...] = jnp.where(iota == 0, total, total * (1.0 / (2 * _NPAIR)))
        pltpu.sync_copy(outv_v, out_hbm)


@jax.jit
def kernel(embedding, label):
    out = pl.kernel(
        _sc_loss,
        out_type=jax.ShapeDtypeStruct((_L,), jnp.float32),
        mesh=plsc.VectorSubcoreMesh(core_axis_name="c", subcore_axis_name="s"),
        compiler_params=pltpu.CompilerParams(needs_layout_passes=False),
        scratch_types=[
            pltpu.VMEM((_N,), jnp.int32),
            pltpu.VMEM((_BUF,), jnp.int32),
            pltpu.VMEM((_BUF,), jnp.int32),
            pltpu.VMEM_SHARED((2 * _NPAIR,), jnp.int32),
            pltpu.VMEM_SHARED((_NW, _L), jnp.float32),
            pltpu.VMEM((2 * _NPAIR,), jnp.int32),
            pltpu.VMEM((4 * _NPAIR,), jnp.int32),
            pltpu.VMEM((_PW, _D), jnp.float32),
            pltpu.VMEM((_PW, _D), jnp.float32),
            pltpu.VMEM((_PW, _D), jnp.float32),
            pltpu.VMEM((_PW, _D), jnp.float32),
            pltpu.VMEM((_L,), jnp.float32),
            pltpu.VMEM((_NW, _L), jnp.float32),
            pltpu.VMEM((_L,), jnp.float32),
            pltpu.SMEM((2,), jnp.int32),
            pltpu.SemaphoreType.DMA,
        ],
    )(embedding, label)
    return (out[1], out[0], (2 * _NPAIR,))
